# baseline (speedup 1.0000x reference)
import jax
import jax.numpy as jnp
from jax.experimental import pallas as pl
from jax.experimental.pallas import tpu as pltpu

_N = 2048
_F = 256
_LEV = 2
_R = 2
_NOP = _LEV * _R - 1
_NS = _NOP * _N

_BM = 512
_ND = _NS // _BM
_MB = _N // _BM
_NST = _ND + _MB


def _fused_kernel(c_ref, rv_ref, d_ref, adj_ref, xbf_ref, h0_ref, wbf_ref,
                  o_ref, dv_ref, zx_ref, acc_ref):
    p = pl.program_id(0)
    qs = jnp.clip(p - 1, 0, _NST - 1)
    jc = jnp.clip(p - 5, 0, _NS // _BM - 1)
    ic = jc // _MB
    mc = jc % _MB

    res_s = jnp.dot(dv_ref[pl.ds(qs * _BM, _BM), :], xbf_ref[...],
                    preferred_element_type=jnp.float32)
    res_c = jnp.dot(dv_ref[pl.ds(ic * _N + mc * _BM, _BM), :],
                    zx_ref[pl.ds(ic * _N, _N), :],
                    preferred_element_type=jnp.float32)

    @pl.when((p >= 1) & (qs < _ND))
    def _():
        zx_ref[pl.ds(qs * _BM, _BM), :] = (
            (c_ref[0] * rv_ref[...]) * res_s).astype(jnp.bfloat16)

    @pl.when((p >= 5) & (p < 5 + _MB))
    def _():
        acc_ref[pl.ds(mc * _BM, _BM), :] = res_c

    @pl.when((p >= 5 + _MB) & (p < 5 + 2 * _MB))
    def _():
        acc_ref[pl.ds(mc * _BM, _BM), :] += res_c

    @pl.when(p >= 5 + 2 * _MB)
    def _():
        s = (c_ref[3] * (acc_ref[pl.ds(mc * _BM, _BM), :] + res_c
                         + c_ref[1] * res_s)
             + c_ref[2] * h0_ref[...])
        o_ref[...] = (c_ref[4] * jnp.dot(s.astype(jnp.bfloat16), wbf_ref[...],
                                         preferred_element_type=jnp.float32)
                      + c_ref[5] * s)

    @pl.when(p < _NST)
    def _():
        blk = jnp.where(p < _ND, d_ref[0], adj_ref[...]).astype(jnp.bfloat16)
        dv_ref[pl.ds(p * _BM, _BM), :] = blk


def kernel(input, adj, d_list, h0, weight, lamda, alpha, l, gamma):
    rv2 = jax.random.uniform(jax.random.key(42), (_LEV * _R * _N, 1),
                             dtype=jnp.float32)[_N:]
    theta = jnp.log(lamda / l + 1)
    g = jnp.asarray(gamma, jnp.float32)
    a = jnp.asarray(alpha, jnp.float32)
    t = jnp.asarray(theta, jnp.float32)
    c = jnp.stack([g, 1 - g, a, 1 - a, t, 1 - t]).astype(jnp.float32)
    xbf = input.astype(jnp.bfloat16)
    wbf = weight.astype(jnp.bfloat16)

    out = pl.pallas_call(
        _fused_kernel,
        grid=(_NST + 1,),
        in_specs=[
            pl.BlockSpec(memory_space=pltpu.SMEM),
            pl.BlockSpec((_BM, 1),
                         lambda p: (jnp.minimum(jnp.clip(p - 1, 0, _NST - 1),
                                                _ND - 1), 0)),
            pl.BlockSpec((1, _BM, _N),
                         lambda p: (1 + jnp.minimum(p, _ND - 1) // _MB,
                                    jnp.minimum(p, _ND - 1) % _MB, 0)),
            pl.BlockSpec((_BM, _N),
                         lambda p: (jnp.clip(p - _ND, 0, _MB - 1), 0)),
            pl.BlockSpec((_N, _F), lambda p: (0, 0)),
            pl.BlockSpec((_BM, _F),
                         lambda p: (jnp.clip(p - (5 + 2 * _MB), 0, _MB - 1), 0)),
            pl.BlockSpec((_F, _F), lambda p: (0, 0)),
        ],
        out_specs=pl.BlockSpec(
            (_BM, _F),
            lambda p: (jnp.clip(p - (5 + 2 * _MB), 0, _MB - 1), 0)),
        out_shape=jax.ShapeDtypeStruct((_N, _F), jnp.float32),
        compiler_params=pltpu.CompilerParams(vmem_limit_bytes=67_000_000),
        scratch_shapes=[
            pltpu.VMEM((_NS + _N, _N), jnp.bfloat16),
            pltpu.VMEM((_NS, _F), jnp.bfloat16),
            pltpu.VMEM((_N, _F), jnp.float32),
        ],
    )(c, rv2, d_list, adj, xbf, h0, wbf)
    return out


# speedup vs baseline: 1.1234x; 1.1234x over previous
import jax
import jax.numpy as jnp
from jax.experimental import pallas as pl
from jax.experimental.pallas import tpu as pltpu

_N = 2048
_F = 256
_LEV = 2
_R = 2
_NOP = _LEV * _R - 1
_NS = _NOP * _N

_BM1 = 512
_BM2 = 512
_P1 = _NS // _BM1
_P2 = _N // _BM2
_NRB = _N // _BM1


def _fused_kernel(c_ref, rv_ref, d_ref, adj_ref, x_ref, h0_ref, w_ref, o_ref,
                  dv_ref, z_ref, xbf_ref):
    p = pl.program_id(0)

    @pl.when(p == 0)
    def _():
        xbf_ref[...] = x_ref[...].astype(jnp.bfloat16)

    @pl.when(p < _P1)
    def _():
        blk = d_ref[0].astype(jnp.bfloat16)
        row = p * _BM1
        dv_ref[pl.ds(row, _BM1), :] = blk
        zblk = (c_ref[0] * rv_ref[...]) * jnp.dot(
            blk, xbf_ref[...], preferred_element_type=jnp.float32)
        z_ref[pl.ds(row, _BM1), :] = zblk.astype(jnp.bfloat16)

    @pl.when(p >= _P1)
    def _():
        m = p - _P1
        acc = c_ref[1] * jnp.dot(adj_ref[...].astype(jnp.bfloat16),
                                 xbf_ref[...],
                                 preferred_element_type=jnp.float32)
        for i in range(_NOP):
            dblk = dv_ref[pl.ds(i * _N + m * _BM2, _BM2), :]
            acc += jnp.dot(dblk, z_ref[pl.ds(i * _N, _N), :],
                           preferred_element_type=jnp.float32)
        s = c_ref[3] * acc + c_ref[2] * h0_ref[...]
        o_ref[...] = (c_ref[4] * jnp.dot(s, w_ref[...],
                                         preferred_element_type=jnp.float32)
                      + c_ref[5] * s)


def kernel(input, adj, d_list, h0, weight, lamda, alpha, l, gamma):
    x = input
    rv2 = jax.random.uniform(jax.random.key(42), (_LEV * _R * _N, 1),
                             dtype=jnp.float32)[_N:]
    theta = jnp.log(lamda / l + 1)
    g = jnp.asarray(gamma, jnp.float32)
    a = jnp.asarray(alpha, jnp.float32)
    t = jnp.asarray(theta, jnp.float32)
    c = jnp.stack([g, 1 - g, a, 1 - a, t, 1 - t]).astype(jnp.float32)

    out = pl.pallas_call(
        _fused_kernel,
        grid=(_P1 + _P2,),
        in_specs=[
            pl.BlockSpec(memory_space=pltpu.SMEM),
            pl.BlockSpec((_BM1, 1),
                         lambda p: (jnp.minimum(p, _P1 - 1), 0)),
            pl.BlockSpec((1, _BM1, _N),
                         lambda p: (1 + jnp.minimum(p, _P1 - 1) // _NRB,
                                    jnp.minimum(p, _P1 - 1) % _NRB, 0)),
            pl.BlockSpec((_BM2, _N),
                         lambda p: (jnp.maximum(p - _P1, 0), 0)),
            pl.BlockSpec((_N, _F), lambda p: (0, 0)),
            pl.BlockSpec((_BM2, _F),
                         lambda p: (jnp.maximum(p - _P1, 0), 0)),
            pl.BlockSpec((_F, _F), lambda p: (0, 0)),
        ],
        out_specs=pl.BlockSpec((_BM2, _F),
                               lambda p: (jnp.maximum(p - _P1, 0), 0)),
        out_shape=jax.ShapeDtypeStruct((_N, _F), jnp.float32),
        compiler_params=pltpu.CompilerParams(vmem_limit_bytes=67_000_000),
        scratch_shapes=[
            pltpu.VMEM((_NS, _N), jnp.bfloat16),
            pltpu.VMEM((_NS, _F), jnp.bfloat16),
            pltpu.VMEM((_N, _F), jnp.bfloat16),
        ],
    )(c, rv2, d_list, adj, x, h0, weight)
    return out


# speedup vs baseline: 1.1578x; 1.0306x over previous
import jax
import jax.numpy as jnp
from jax.experimental import pallas as pl
from jax.experimental.pallas import tpu as pltpu

_N = 2048
_F = 256
_LEV = 2
_R = 2
_NOP = _LEV * _R - 1
_NS = _NOP * _N

_BM1 = 1024
_BM2 = 512
_P1 = _NS // _BM1
_P2 = _N // _BM2
_NRB = _N // _BM1


def _fused_kernel(c_ref, rv_ref, d_ref, adj_ref, x_ref, h0_ref, w_ref, o_ref,
                  dv_ref, z_ref, xbf_ref):
    p = pl.program_id(0)

    @pl.when(p == 0)
    def _():
        xbf_ref[...] = x_ref[...].astype(jnp.bfloat16)

    @pl.when(p < _P1)
    def _():
        blk = d_ref[0].astype(jnp.bfloat16)
        row = p * _BM1
        dv_ref[pl.ds(row, _BM1), :] = blk
        zblk = (c_ref[0] * rv_ref[...]) * jnp.dot(
            blk, xbf_ref[...], preferred_element_type=jnp.float32)
        z_ref[pl.ds(row, _BM1), :] = zblk.astype(jnp.bfloat16)

    @pl.when(p >= _P1)
    def _():
        m = p - _P1
        acc = c_ref[1] * jnp.dot(adj_ref[...].astype(jnp.bfloat16),
                                 xbf_ref[...],
                                 preferred_element_type=jnp.float32)
        for i in range(_NOP):
            dblk = dv_ref[pl.ds(i * _N + m * _BM2, _BM2), :]
            acc += jnp.dot(dblk, z_ref[pl.ds(i * _N, _N), :],
                           preferred_element_type=jnp.float32)
        s = c_ref[3] * acc + c_ref[2] * h0_ref[...]
        o_ref[...] = (c_ref[4] * jnp.dot(s, w_ref[...],
                                         preferred_element_type=jnp.float32)
                      + c_ref[5] * s)


def kernel(input, adj, d_list, h0, weight, lamda, alpha, l, gamma):
    x = input
    rv2 = jax.random.uniform(jax.random.key(42), (_LEV * _R * _N, 1),
                             dtype=jnp.float32)[_N:]
    theta = jnp.log(lamda / l + 1)
    g = jnp.asarray(gamma, jnp.float32)
    a = jnp.asarray(alpha, jnp.float32)
    t = jnp.asarray(theta, jnp.float32)
    c = jnp.stack([g, 1 - g, a, 1 - a, t, 1 - t]).astype(jnp.float32)

    out = pl.pallas_call(
        _fused_kernel,
        grid=(_P1 + _P2,),
        in_specs=[
            pl.BlockSpec(memory_space=pltpu.SMEM),
            pl.BlockSpec((_BM1, 1),
                         lambda p: (jnp.minimum(p, _P1 - 1), 0)),
            pl.BlockSpec((1, _BM1, _N),
                         lambda p: (1 + jnp.minimum(p, _P1 - 1) // _NRB,
                                    jnp.minimum(p, _P1 - 1) % _NRB, 0)),
            pl.BlockSpec((_BM2, _N),
                         lambda p: (jnp.maximum(p - _P1, 0), 0)),
            pl.BlockSpec((_N, _F), lambda p: (0, 0)),
            pl.BlockSpec((_BM2, _F),
                         lambda p: (jnp.maximum(p - _P1, 0), 0)),
            pl.BlockSpec((_F, _F), lambda p: (0, 0)),
        ],
        out_specs=pl.BlockSpec((_BM2, _F),
                               lambda p: (jnp.maximum(p - _P1, 0), 0)),
        out_shape=jax.ShapeDtypeStruct((_N, _F), jnp.float32),
        compiler_params=pltpu.CompilerParams(vmem_limit_bytes=67_000_000),
        scratch_shapes=[
            pltpu.VMEM((_NS, _N), jnp.bfloat16),
            pltpu.VMEM((_NS, _F), jnp.bfloat16),
            pltpu.VMEM((_N, _F), jnp.bfloat16),
        ],
    )(c, rv2, d_list, adj, x, h0, weight)
    return out
